# baseline (speedup 1.0000x reference)
import jax
import jax.numpy as jnp
from jax import lax
from jax.experimental import pallas as pl
from jax.experimental.pallas import tpu as pltpu
from jax.experimental.pallas import tpu_sc as plsc

N = 50000
E = 800000
NS = 50048
EP = 819200
NCH = EP // 128
RPT = NS // 16
CH = 8

_mesh = plsc.VectorSubcoreMesh(core_axis_name="c", subcore_axis_name="s")
_params = pltpu.CompilerParams(use_tc_tiling_on_sc=False)


def _deg_kernel(dst_hbm, ones_hbm, zeros_hbm, out_hbm, dst_v, ones_v, acc_sh):
    c = lax.axis_index("c")
    s = lax.axis_index("s")
    w = c * 16 + s
    pltpu.sync_copy(ones_hbm, ones_v)
    pltpu.sync_copy(zeros_hbm, acc_sh.at[pl.ds(s * RPT, RPT)])
    plsc.subcore_barrier()
    base = w * (NCH // 32)

    def body(b, carry):
        row0 = pl.multiple_of(base + b * CH, 8)
        pltpu.sync_copy(dst_hbm.at[pl.ds(row0, CH)], dst_v)
        for j in range(CH):
            pltpu.sync_copy(ones_v, acc_sh.at[dst_v.at[j]], add=True)
        return carry

    lax.fori_loop(0, NCH // 32 // CH, body, 0)
    plsc.subcore_barrier()
    off = pl.multiple_of(c * NS + s * RPT, 8)
    pltpu.sync_copy(acc_sh.at[pl.ds(s * RPT, RPT)], out_hbm.at[pl.ds(off, RPT)])


def _make_deg():
    return pl.kernel(
        _deg_kernel,
        out_type=jax.ShapeDtypeStruct((2 * NS,), jnp.float32),
        mesh=_mesh,
        compiler_params=_params,
        scratch_types=[
            pltpu.VMEM((CH, 128), jnp.int32),
            pltpu.VMEM((128,), jnp.float32),
            pltpu.VMEM_SHARED((NS,), jnp.float32),
        ],
    )


def _agg_edge_kernel(table_hbm, src_hbm, dst_hbm, zeros_hbm, out_hbm,
                     src_v, dst_v, rows_v, sem, acc_sh):
    c = lax.axis_index("c")
    s = lax.axis_index("s")
    w = c * 16 + s
    pltpu.sync_copy(zeros_hbm, acc_sh.at[pl.ds(s * RPT, RPT)])
    plsc.subcore_barrier()
    base = w * (NCH // 32)

    def body(b, carry):
        row0 = pl.multiple_of(base + b * CH, 8)
        pltpu.sync_copy(src_hbm.at[pl.ds(row0, CH)], src_v)
        pltpu.sync_copy(dst_hbm.at[pl.ds(row0, CH)], dst_v)
        cps = [pltpu.async_copy(table_hbm.at[src_v.at[j]], rows_v.at[j], sem)
               for j in range(CH)]
        for cp in cps:
            cp.wait()
        for j in range(CH):
            pltpu.sync_copy(rows_v.at[j], acc_sh.at[dst_v.at[j]], add=True)
        return carry

    lax.fori_loop(0, NCH // 32 // CH, body, 0)
    plsc.subcore_barrier()
    pltpu.sync_copy(acc_sh.at[pl.ds(s * RPT, RPT)],
                    out_hbm.at[c].at[pl.ds(s * RPT, RPT)])


def _make_agg_edge():
    return pl.kernel(
        _agg_edge_kernel,
        out_type=jax.ShapeDtypeStruct((2, NS, 16), jnp.float32),
        mesh=_mesh,
        compiler_params=_params,
        scratch_types=[
            pltpu.VMEM((CH, 128), jnp.int32),
            pltpu.VMEM((CH, 128), jnp.int32),
            pltpu.VMEM((CH, 128, 16), jnp.float32),
            pltpu.SemaphoreType.DMA,
            pltpu.VMEM_SHARED((NS, 16), jnp.float32),
        ],
    )


def _agg_quad_kernel(table_hbm, src_hbm, dst_hbm, zeros_hbm, out_hbm,
                     src_v, dst_v, rows_v, sem, acc_sh):
    c = lax.axis_index("c")
    s = lax.axis_index("s")
    base = s * (NCH // 16)

    for r in range(2):
        q = 2 * c + r
        src_q = src_hbm.at[q]
        pltpu.sync_copy(zeros_hbm, acc_sh.at[pl.ds(s * RPT, RPT)])
        plsc.subcore_barrier()

        def body(b, carry):
            row0 = pl.multiple_of(base + b * CH, 8)
            pltpu.sync_copy(src_q.at[pl.ds(row0, CH)], src_v)
            pltpu.sync_copy(dst_hbm.at[pl.ds(row0, CH)], dst_v)
            cps = [pltpu.async_copy(table_hbm.at[src_v.at[j]], rows_v.at[j],
                                    sem) for j in range(CH)]
            for cp in cps:
                cp.wait()
            for j in range(CH):
                pltpu.sync_copy(rows_v.at[j], acc_sh.at[dst_v.at[j]], add=True)
            return carry

        lax.fori_loop(0, NCH // 16 // CH, body, 0)
        plsc.subcore_barrier()
        pltpu.sync_copy(acc_sh.at[pl.ds(s * RPT, RPT)],
                        out_hbm.at[q].at[pl.ds(s * RPT, RPT)])


def _make_agg_quad():
    return pl.kernel(
        _agg_quad_kernel,
        out_type=jax.ShapeDtypeStruct((4, NS, 16), jnp.float32),
        mesh=_mesh,
        compiler_params=_params,
        scratch_types=[
            pltpu.VMEM((CH, 128), jnp.int32),
            pltpu.VMEM((CH, 128), jnp.int32),
            pltpu.VMEM((CH, 128, 16), jnp.float32),
            pltpu.SemaphoreType.DMA,
            pltpu.VMEM_SHARED((NS, 16), jnp.float32),
        ],
    )


NS8 = NS // 8
_R8 = NS8 // 2
_LM = NS // 128


def _dinv_body(deg_ref, dinv_ref):
    rid = (lax.broadcasted_iota(jnp.int32, (_LM, 128), 0) * 128
           + lax.broadcasted_iota(jnp.int32, (_LM, 128), 1))
    deg = deg_ref[0] + deg_ref[1] + 1.0
    dinv_ref[...] = jnp.where(rid < N, lax.rsqrt(deg), 0.0)


def _b1_body(t_ref, w_ref, b_ref, rep_ref, out_ref):
    h = jnp.maximum(
        jnp.dot(t_ref[...], w_ref[...], preferred_element_type=jnp.float32)
        + b_ref[...], 0.0)
    out_ref[...] = h * rep_ref[...]


def _b2_body(t_ref, w1_ref, b1_ref, rep_ref, w2_ref, out_ref):
    h2 = jnp.maximum(
        jnp.dot(t_ref[...], w1_ref[...], preferred_element_type=jnp.float32)
        + b1_ref[...], 0.0)
    out_ref[...] = jnp.dot(h2 * rep_ref[...], w2_ref[...],
                           preferred_element_type=jnp.float32)


def _b3_body(a0_ref, a1_ref, m_ref, rep_ref, b_ref, out_ref):
    out_ref[...] = (rep_ref[...] * (a0_ref[...] + a1_ref[...] + m_ref[...])
                    + b_ref[...])


def _blk_spec(d):
    return pl.BlockSpec((_R8, d), lambda i: (i, 0))


def _full_spec(shape):
    return pl.BlockSpec(shape, lambda i: tuple(0 for _ in shape))


def kernel(x, edge_index, W0, b0, W1, b1, W2, b2):
    f32 = jnp.float32
    src = edge_index[0].astype(jnp.int32)
    dst = edge_index[1].astype(jnp.int32)
    pad_vals = N + (jnp.arange(EP - E, dtype=jnp.int32) % 48)
    srcP = jnp.concatenate([src, pad_vals]).reshape(NCH, 128)
    dstP = jnp.concatenate([dst, pad_vals]).reshape(NCH, 128)
    srcO = jnp.stack([srcP * 4 + q for q in range(4)])
    x_pad = jnp.pad(x, ((0, NS - N), (0, 0)))
    zeros16 = jnp.zeros((RPT, 16), f32)
    zeros1 = jnp.zeros((RPT,), f32)
    ones128 = jnp.ones((128,), f32)
    eye8 = jnp.eye(8, dtype=f32)
    W0b = jnp.kron(eye8, W0)
    W1b = jnp.kron(eye8, W1)
    W2b = jnp.kron(eye8, jnp.pad(W2, ((0, 0), (0, 1))))
    b0r = jnp.tile(b0, 8)[None, :]
    b1r = jnp.tile(b1, 8)[None, :]
    b2r = jnp.tile(jnp.pad(b2, (0, 1)), 8)[None, :]

    deg1d = _make_deg()(dstP, ones128, zeros1)
    dinv_lm = pl.pallas_call(
        _dinv_body,
        out_shape=jax.ShapeDtypeStruct((_LM, 128), f32),
    )(deg1d.reshape(2, _LM, 128))
    dinv1 = dinv_lm.reshape(NS)
    d8 = dinv1.reshape(NS8, 8)
    rep512 = jnp.repeat(d8, 64, axis=1)
    rep128 = jnp.repeat(d8, 16, axis=1)

    xs16 = x_pad * dinv1[:, None]
    agg0 = _make_agg_edge()(xs16, srcP, dstP, zeros16)

    tin = ((agg0[0] + agg0[1] + xs16) * dinv1[:, None]).reshape(NS8, 128)
    hsp = pl.pallas_call(
        _b1_body,
        grid=(2,),
        in_specs=[_blk_spec(128), _full_spec((128, 512)),
                  _full_spec((1, 512)), _blk_spec(512)],
        out_specs=_blk_spec(512),
        out_shape=jax.ShapeDtypeStruct((NS8, 512), f32),
    )(tin, W0b, b0r, rep512)

    agg1 = _make_agg_quad()(hsp.reshape(4 * NS, 16), srcO, dstP, zeros16)

    hs64 = hsp.reshape(NS, 64)
    agg1cat = jnp.concatenate([agg1[q] for q in range(4)], axis=1)
    t1 = ((agg1cat + hs64) * dinv1[:, None]).reshape(NS8, 512)
    mp = pl.pallas_call(
        _b2_body,
        grid=(2,),
        in_specs=[_blk_spec(512), _full_spec((512, 512)),
                  _full_spec((1, 512)), _blk_spec(512),
                  _full_spec((512, 128))],
        out_specs=_blk_spec(128),
        out_shape=jax.ShapeDtypeStruct((NS8, 128), f32),
    )(t1, W1b, b1r, rep512, W2b)

    agg2 = _make_agg_edge()(mp.reshape(NS, 16), srcP, dstP, zeros16)

    outp = pl.pallas_call(
        _b3_body,
        grid=(2,),
        in_specs=[_blk_spec(128), _blk_spec(128), _blk_spec(128),
                  _blk_spec(128), _full_spec((1, 128))],
        out_specs=_blk_spec(128),
        out_shape=jax.ShapeDtypeStruct((NS8, 128), f32),
    )(agg2[0].reshape(NS8, 128), agg2[1].reshape(NS8, 128), mp, rep128, b2r)

    return outp.reshape(NS, 16)[:N, :15]


# speedup vs baseline: 1.0911x; 1.0911x over previous
import jax
import jax.numpy as jnp
from jax import lax
from jax.experimental import pallas as pl
from jax.experimental.pallas import tpu as pltpu
from jax.experimental.pallas import tpu_sc as plsc

N = 50000
E = 800000
NS = 50048
EP = 851968
NCH = EP // 128
RPT = NS // 16
CH = 8

_mesh = plsc.VectorSubcoreMesh(core_axis_name="c", subcore_axis_name="s")
_params = pltpu.CompilerParams(use_tc_tiling_on_sc=False)


def _deg_kernel(dst_hbm, ones_hbm, zeros_hbm, out_hbm, dst_v, ones_v, acc_sh):
    c = lax.axis_index("c")
    s = lax.axis_index("s")
    w = c * 16 + s
    pltpu.sync_copy(ones_hbm, ones_v)
    pltpu.sync_copy(zeros_hbm, acc_sh.at[pl.ds(s * RPT, RPT)])
    plsc.subcore_barrier()
    base = w * (NCH // 32)

    def body(b, carry):
        row0 = pl.multiple_of(base + b * CH, 8)
        pltpu.sync_copy(dst_hbm.at[pl.ds(row0, CH)], dst_v)
        for j in range(CH):
            pltpu.sync_copy(ones_v, acc_sh.at[dst_v.at[j]], add=True)
        return carry

    lax.fori_loop(0, NCH // 32 // CH, body, 0)
    plsc.subcore_barrier()
    off = pl.multiple_of(c * NS + s * RPT, 8)
    pltpu.sync_copy(acc_sh.at[pl.ds(s * RPT, RPT)], out_hbm.at[pl.ds(off, RPT)])


def _make_deg():
    return pl.kernel(
        _deg_kernel,
        out_type=jax.ShapeDtypeStruct((2 * NS,), jnp.float32),
        mesh=_mesh,
        compiler_params=_params,
        scratch_types=[
            pltpu.VMEM((CH, 128), jnp.int32),
            pltpu.VMEM((128,), jnp.float32),
            pltpu.VMEM_SHARED((NS,), jnp.float32),
        ],
    )


def _load_idx(src_hbm, dst_hbm, row0, src_v, dst_v, qmul):
    pltpu.sync_copy(src_hbm.at[pl.ds(row0, CH)], src_v)
    pltpu.sync_copy(dst_hbm.at[pl.ds(row0, CH)], dst_v)
    if qmul is not None:
        for j in range(CH):
            for k in range(8):
                sl = pl.ds(16 * k, 16)
                src_v[j, sl] = src_v[j, sl] * 4 + qmul


def _fire_g(table_hbm, src_v, rows_v, gsem):
    for j in range(CH):
        pltpu.async_copy(table_hbm.at[src_v.at[j]], rows_v.at[j], gsem)


def _drain_g(table_hbm, src_v, rows_v, gsem):
    for j in range(CH):
        pltpu.make_async_copy(table_hbm.at[src_v.at[j]], rows_v.at[j],
                              gsem).wait()


def _fire_s(acc_sh, dst_v, rows_v, ssem):
    for j in range(CH):
        pltpu.async_copy(rows_v.at[j], acc_sh.at[dst_v.at[j]], ssem, add=True)


def _drain_s(acc_sh, dst_v, rows_v, ssem):
    for j in range(CH):
        pltpu.make_async_copy(rows_v.at[j], acc_sh.at[dst_v.at[j]],
                              ssem).wait()


def _agg_pass(table_hbm, src_hbm, dst_hbm, acc_sh, base, nb, qmul,
              sA, dA, rA, gsA, ssA, sB, dB, rB, gsB, ssB):
    r0 = pl.multiple_of(base, 8)
    _load_idx(src_hbm, dst_hbm, r0, sA, dA, qmul)
    _fire_g(table_hbm, sA, rA, gsA)
    _drain_g(table_hbm, sA, rA, gsA)
    _fire_s(acc_sh, dA, rA, ssA)
    r1 = pl.multiple_of(base + CH, 8)
    _load_idx(src_hbm, dst_hbm, r1, sB, dB, qmul)
    _fire_g(table_hbm, sB, rB, gsB)

    def handle(b, sP, dP, rP, gsP, ssP, sQ, dQ, rQ, gsQ, ssQ):
        _drain_g(table_hbm, sP, rP, gsP)
        _fire_s(acc_sh, dP, rP, ssP)
        _drain_s(acc_sh, dQ, rQ, ssQ)
        rn = pl.multiple_of(base + (b + 1) * CH, 8)
        _load_idx(src_hbm, dst_hbm, rn, sQ, dQ, qmul)
        _fire_g(table_hbm, sQ, rQ, gsQ)

    def body(i, carry):
        b = 1 + 2 * i
        handle(b, sB, dB, rB, gsB, ssB, sA, dA, rA, gsA, ssA)
        handle(b + 1, sA, dA, rA, gsA, ssA, sB, dB, rB, gsB, ssB)
        return carry

    lax.fori_loop(0, (nb - 2) // 2, body, 0)
    _drain_g(table_hbm, sB, rB, gsB)
    _fire_s(acc_sh, dB, rB, ssB)
    _drain_s(acc_sh, dA, rA, ssA)
    _drain_s(acc_sh, dB, rB, ssB)


def _agg_edge_kernel(table_hbm, src_hbm, dst_hbm, zeros_hbm, out_hbm,
                     sA, dA, rA, sB, dB, rB, gsA, ssA, gsB, ssB, acc_sh):
    c = lax.axis_index("c")
    s = lax.axis_index("s")
    w = c * 16 + s
    pltpu.sync_copy(zeros_hbm, acc_sh.at[pl.ds(s * RPT, RPT)])
    plsc.subcore_barrier()
    _agg_pass(table_hbm, src_hbm, dst_hbm, acc_sh, w * (NCH // 32),
              NCH // 32 // CH, None,
              sA, dA, rA, gsA, ssA, sB, dB, rB, gsB, ssB)
    plsc.subcore_barrier()
    pltpu.sync_copy(acc_sh.at[pl.ds(s * RPT, RPT)],
                    out_hbm.at[c].at[pl.ds(s * RPT, RPT)])


def _make_agg_edge():
    return pl.kernel(
        _agg_edge_kernel,
        out_type=jax.ShapeDtypeStruct((2, NS, 16), jnp.float32),
        mesh=_mesh,
        compiler_params=_params,
        scratch_types=[
            pltpu.VMEM((CH, 128), jnp.int32),
            pltpu.VMEM((CH, 128), jnp.int32),
            pltpu.VMEM((CH, 128, 16), jnp.float32),
            pltpu.VMEM((CH, 128), jnp.int32),
            pltpu.VMEM((CH, 128), jnp.int32),
            pltpu.VMEM((CH, 128, 16), jnp.float32),
            pltpu.SemaphoreType.DMA,
            pltpu.SemaphoreType.DMA,
            pltpu.SemaphoreType.DMA,
            pltpu.SemaphoreType.DMA,
            pltpu.VMEM_SHARED((NS, 16), jnp.float32),
        ],
    )


def _agg_quad_kernel(table_hbm, src_hbm, dst_hbm, zeros_hbm, out_hbm,
                     sA, dA, rA, sB, dB, rB, gsA, ssA, gsB, ssB, acc_sh):
    c = lax.axis_index("c")
    s = lax.axis_index("s")
    base = s * (NCH // 16)

    for r in range(2):
        q = 2 * c + r
        pltpu.sync_copy(zeros_hbm, acc_sh.at[pl.ds(s * RPT, RPT)])
        plsc.subcore_barrier()
        _agg_pass(table_hbm, src_hbm, dst_hbm, acc_sh, base,
                  NCH // 16 // CH, q,
                  sA, dA, rA, gsA, ssA, sB, dB, rB, gsB, ssB)
        plsc.subcore_barrier()
        pltpu.sync_copy(acc_sh.at[pl.ds(s * RPT, RPT)],
                        out_hbm.at[q].at[pl.ds(s * RPT, RPT)])


def _make_agg_quad():
    return pl.kernel(
        _agg_quad_kernel,
        out_type=jax.ShapeDtypeStruct((4, NS, 16), jnp.float32),
        mesh=_mesh,
        compiler_params=_params,
        scratch_types=[
            pltpu.VMEM((CH, 128), jnp.int32),
            pltpu.VMEM((CH, 128), jnp.int32),
            pltpu.VMEM((CH, 128, 16), jnp.float32),
            pltpu.VMEM((CH, 128), jnp.int32),
            pltpu.VMEM((CH, 128), jnp.int32),
            pltpu.VMEM((CH, 128, 16), jnp.float32),
            pltpu.SemaphoreType.DMA,
            pltpu.SemaphoreType.DMA,
            pltpu.SemaphoreType.DMA,
            pltpu.SemaphoreType.DMA,
            pltpu.VMEM_SHARED((NS, 16), jnp.float32),
        ],
    )


NS8 = NS // 8
_R8 = NS8 // 2
_LM = NS // 128


def _dinv_body(deg_ref, dinv_ref):
    rid = (lax.broadcasted_iota(jnp.int32, (_LM, 128), 0) * 128
           + lax.broadcasted_iota(jnp.int32, (_LM, 128), 1))
    deg = deg_ref[0] + deg_ref[1] + 1.0
    dinv_ref[...] = jnp.where(rid < N, lax.rsqrt(deg), 0.0)


def _b1_body(t_ref, w_ref, b_ref, rep_ref, out_ref):
    h = jnp.maximum(
        jnp.dot(t_ref[...], w_ref[...], preferred_element_type=jnp.float32)
        + b_ref[...], 0.0)
    out_ref[...] = h * rep_ref[...]


def _b2_body(t_ref, w1_ref, b1_ref, rep_ref, w2_ref, out_ref):
    h2 = jnp.maximum(
        jnp.dot(t_ref[...], w1_ref[...], preferred_element_type=jnp.float32)
        + b1_ref[...], 0.0)
    out_ref[...] = jnp.dot(h2 * rep_ref[...], w2_ref[...],
                           preferred_element_type=jnp.float32)


def _b3_body(a0_ref, a1_ref, m_ref, rep_ref, b_ref, out_ref):
    out_ref[...] = (rep_ref[...] * (a0_ref[...] + a1_ref[...] + m_ref[...])
                    + b_ref[...])


def _blk_spec(d):
    return pl.BlockSpec((_R8, d), lambda i: (i, 0))


def _full_spec(shape):
    return pl.BlockSpec(shape, lambda i: tuple(0 for _ in shape))


def kernel(x, edge_index, W0, b0, W1, b1, W2, b2):
    f32 = jnp.float32
    src = edge_index[0].astype(jnp.int32)
    dst = edge_index[1].astype(jnp.int32)
    pad_vals = N + (jnp.arange(EP - E, dtype=jnp.int32) % 48)
    srcP = jnp.concatenate([src, pad_vals]).reshape(NCH, 128)
    dstP = jnp.concatenate([dst, pad_vals]).reshape(NCH, 128)
    x_pad = jnp.pad(x, ((0, NS - N), (0, 0)))
    zeros16 = jnp.zeros((RPT, 16), f32)
    zeros1 = jnp.zeros((RPT,), f32)
    ones128 = jnp.ones((128,), f32)
    eye8 = jnp.eye(8, dtype=f32)
    W0b = jnp.kron(eye8, W0)
    W1b = jnp.kron(eye8, W1)
    W2b = jnp.kron(eye8, jnp.pad(W2, ((0, 0), (0, 1))))
    b0r = jnp.tile(b0, 8)[None, :]
    b1r = jnp.tile(b1, 8)[None, :]
    b2r = jnp.tile(jnp.pad(b2, (0, 1)), 8)[None, :]

    deg1d = _make_deg()(dstP, ones128, zeros1)
    dinv_lm = pl.pallas_call(
        _dinv_body,
        out_shape=jax.ShapeDtypeStruct((_LM, 128), f32),
    )(deg1d.reshape(2, _LM, 128))
    dinv1 = dinv_lm.reshape(NS)
    d8 = dinv1.reshape(NS8, 8)
    rep512 = jnp.repeat(d8, 64, axis=1)
    rep128 = jnp.repeat(d8, 16, axis=1)

    xs16 = x_pad * dinv1[:, None]
    agg0 = _make_agg_edge()(xs16, srcP, dstP, zeros16)

    tin = ((agg0[0] + agg0[1] + xs16) * dinv1[:, None]).reshape(NS8, 128)
    hsp = pl.pallas_call(
        _b1_body,
        grid=(2,),
        in_specs=[_blk_spec(128), _full_spec((128, 512)),
                  _full_spec((1, 512)), _blk_spec(512)],
        out_specs=_blk_spec(512),
        out_shape=jax.ShapeDtypeStruct((NS8, 512), f32),
    )(tin, W0b, b0r, rep512)

    agg1 = _make_agg_quad()(hsp.reshape(4 * NS, 16), srcP, dstP, zeros16)

    hs64 = hsp.reshape(NS, 64)
    agg1cat = jnp.concatenate([agg1[q] for q in range(4)], axis=1)
    t1 = ((agg1cat + hs64) * dinv1[:, None]).reshape(NS8, 512)
    mp = pl.pallas_call(
        _b2_body,
        grid=(2,),
        in_specs=[_blk_spec(512), _full_spec((512, 512)),
                  _full_spec((1, 512)), _blk_spec(512),
                  _full_spec((512, 128))],
        out_specs=_blk_spec(128),
        out_shape=jax.ShapeDtypeStruct((NS8, 128), f32),
    )(t1, W1b, b1r, rep512, W2b)

    agg2 = _make_agg_edge()(mp.reshape(NS, 16), srcP, dstP, zeros16)

    outp = pl.pallas_call(
        _b3_body,
        grid=(2,),
        in_specs=[_blk_spec(128), _blk_spec(128), _blk_spec(128),
                  _blk_spec(128), _full_spec((1, 128))],
        out_specs=_blk_spec(128),
        out_shape=jax.ShapeDtypeStruct((NS8, 128), f32),
    )(agg2[0].reshape(NS8, 128), agg2[1].reshape(NS8, 128), mp, rep128, b2r)

    return outp.reshape(NS, 16)[:N, :15]


# speedup vs baseline: 1.1827x; 1.0840x over previous
import jax
import jax.numpy as jnp
from jax import lax
from jax.experimental import pallas as pl
from jax.experimental.pallas import tpu as pltpu
from jax.experimental.pallas import tpu_sc as plsc

N = 50000
E = 800000
NS = 50048
EP = 851968
NCH = EP // 128
RPT = NS // 16
CH = 8

_mesh = plsc.VectorSubcoreMesh(core_axis_name="c", subcore_axis_name="s")
_params = pltpu.CompilerParams(use_tc_tiling_on_sc=False)


def _deg_kernel(dst_hbm, ones_hbm, zeros_hbm, out_hbm, dst_v, ones_v, acc_sh):
    c = lax.axis_index("c")
    s = lax.axis_index("s")
    w = c * 16 + s
    pltpu.sync_copy(ones_hbm, ones_v)
    pltpu.sync_copy(zeros_hbm, acc_sh.at[pl.ds(s * RPT, RPT)])
    plsc.subcore_barrier()
    base = w * (NCH // 32)

    def body(b, carry):
        row0 = pl.multiple_of(base + b * CH, 8)
        pltpu.sync_copy(dst_hbm.at[pl.ds(row0, CH)], dst_v)
        for j in range(CH):
            pltpu.sync_copy(ones_v, acc_sh.at[dst_v.at[j]], add=True)
        return carry

    lax.fori_loop(0, NCH // 32 // CH, body, 0)
    plsc.subcore_barrier()
    off = pl.multiple_of(c * NS + s * RPT, 8)
    pltpu.sync_copy(acc_sh.at[pl.ds(s * RPT, RPT)], out_hbm.at[pl.ds(off, RPT)])


def _make_deg():
    return pl.kernel(
        _deg_kernel,
        out_type=jax.ShapeDtypeStruct((2 * NS,), jnp.float32),
        mesh=_mesh,
        compiler_params=_params,
        scratch_types=[
            pltpu.VMEM((CH, 128), jnp.int32),
            pltpu.VMEM((128,), jnp.float32),
            pltpu.VMEM_SHARED((NS,), jnp.float32),
        ],
    )


def _load_idx(src_hbm, dst_hbm, row0, src_v, dst_v, qmul):
    pltpu.sync_copy(src_hbm.at[pl.ds(row0, CH)], src_v)
    pltpu.sync_copy(dst_hbm.at[pl.ds(row0, CH)], dst_v)
    if qmul is not None:
        for j in range(CH):
            for k in range(8):
                sl = pl.ds(16 * k, 16)
                src_v[j, sl] = src_v[j, sl] * 4 + qmul


def _fire_g(table_hbm, src_v, rows_v, gsem):
    for j in range(CH):
        pltpu.async_copy(table_hbm.at[src_v.at[j]], rows_v.at[j], gsem)


def _drain_g(table_hbm, src_v, rows_v, gsem):
    for j in range(CH):
        pltpu.make_async_copy(table_hbm.at[src_v.at[j]], rows_v.at[j],
                              gsem).wait()


def _fire_s(acc_sh, dst_v, rows_v, ssem):
    for j in range(CH):
        pltpu.async_copy(rows_v.at[j], acc_sh.at[dst_v.at[j]], ssem, add=True)


def _drain_s(acc_sh, dst_v, rows_v, ssem):
    for j in range(CH):
        pltpu.make_async_copy(rows_v.at[j], acc_sh.at[dst_v.at[j]],
                              ssem).wait()


def _agg_pass(table_hbm, src_hbm, dst_hbm, acc_sh, base, nb, qmul,
              sA, dA, rA, gsA, ssA, sB, dB, rB, gsB, ssB):
    r0 = pl.multiple_of(base, 8)
    _load_idx(src_hbm, dst_hbm, r0, sA, dA, qmul)
    _fire_g(table_hbm, sA, rA, gsA)
    _drain_g(table_hbm, sA, rA, gsA)
    _fire_s(acc_sh, dA, rA, ssA)
    r1 = pl.multiple_of(base + CH, 8)
    _load_idx(src_hbm, dst_hbm, r1, sB, dB, qmul)
    _fire_g(table_hbm, sB, rB, gsB)

    def handle(b, sP, dP, rP, gsP, ssP, sQ, dQ, rQ, gsQ, ssQ):
        _drain_g(table_hbm, sP, rP, gsP)
        _fire_s(acc_sh, dP, rP, ssP)
        _drain_s(acc_sh, dQ, rQ, ssQ)
        rn = pl.multiple_of(base + (b + 1) * CH, 8)
        _load_idx(src_hbm, dst_hbm, rn, sQ, dQ, qmul)
        _fire_g(table_hbm, sQ, rQ, gsQ)

    def body(i, carry):
        b = 1 + 2 * i
        handle(b, sB, dB, rB, gsB, ssB, sA, dA, rA, gsA, ssA)
        handle(b + 1, sA, dA, rA, gsA, ssA, sB, dB, rB, gsB, ssB)
        return carry

    lax.fori_loop(0, (nb - 2) // 2, body, 0)
    _drain_g(table_hbm, sB, rB, gsB)
    _fire_s(acc_sh, dB, rB, ssB)
    _drain_s(acc_sh, dA, rA, ssA)
    _drain_s(acc_sh, dB, rB, ssB)


def _agg_edge_kernel(table_hbm, src_hbm, dst_hbm, zeros_hbm, out_hbm,
                     sA, dA, rA, sB, dB, rB, gsA, ssA, gsB, ssB, acc_sh):
    c = lax.axis_index("c")
    s = lax.axis_index("s")
    w = c * 16 + s
    pltpu.sync_copy(zeros_hbm, acc_sh.at[pl.ds(s * RPT, RPT)])
    plsc.subcore_barrier()
    _agg_pass(table_hbm, src_hbm, dst_hbm, acc_sh, w * (NCH // 32),
              NCH // 32 // CH, None,
              sA, dA, rA, gsA, ssA, sB, dB, rB, gsB, ssB)
    plsc.subcore_barrier()
    pltpu.sync_copy(acc_sh.at[pl.ds(s * RPT, RPT)],
                    out_hbm.at[c].at[pl.ds(s * RPT, RPT)])


def _make_agg_edge():
    return pl.kernel(
        _agg_edge_kernel,
        out_type=jax.ShapeDtypeStruct((2, NS, 16), jnp.float32),
        mesh=_mesh,
        compiler_params=_params,
        scratch_types=[
            pltpu.VMEM((CH, 128), jnp.int32),
            pltpu.VMEM((CH, 128), jnp.int32),
            pltpu.VMEM((CH, 128, 16), jnp.float32),
            pltpu.VMEM((CH, 128), jnp.int32),
            pltpu.VMEM((CH, 128), jnp.int32),
            pltpu.VMEM((CH, 128, 16), jnp.float32),
            pltpu.SemaphoreType.DMA,
            pltpu.SemaphoreType.DMA,
            pltpu.SemaphoreType.DMA,
            pltpu.SemaphoreType.DMA,
            pltpu.VMEM_SHARED((NS, 16), jnp.float32),
        ],
    )


def _agg_quad_kernel(table_hbm, src_hbm, dst_hbm, zeros_hbm, out_hbm,
                     sA, dA, rA, sB, dB, rB, gsA, ssA, gsB, ssB, acc_sh):
    c = lax.axis_index("c")
    s = lax.axis_index("s")
    base = s * (NCH // 16)

    for r in range(2):
        q = 2 * c + r
        pltpu.sync_copy(zeros_hbm, acc_sh.at[pl.ds(s * RPT, RPT)])
        plsc.subcore_barrier()
        _agg_pass(table_hbm, src_hbm, dst_hbm, acc_sh, base,
                  NCH // 16 // CH, q,
                  sA, dA, rA, gsA, ssA, sB, dB, rB, gsB, ssB)
        plsc.subcore_barrier()
        pltpu.sync_copy(acc_sh.at[pl.ds(s * RPT, RPT)],
                        out_hbm.at[q].at[pl.ds(s * RPT, RPT)])


def _make_agg_quad():
    return pl.kernel(
        _agg_quad_kernel,
        out_type=jax.ShapeDtypeStruct((4, NS, 16), jnp.float32),
        mesh=_mesh,
        compiler_params=_params,
        scratch_types=[
            pltpu.VMEM((CH, 128), jnp.int32),
            pltpu.VMEM((CH, 128), jnp.int32),
            pltpu.VMEM((CH, 128, 16), jnp.float32),
            pltpu.VMEM((CH, 128), jnp.int32),
            pltpu.VMEM((CH, 128), jnp.int32),
            pltpu.VMEM((CH, 128, 16), jnp.float32),
            pltpu.SemaphoreType.DMA,
            pltpu.SemaphoreType.DMA,
            pltpu.SemaphoreType.DMA,
            pltpu.SemaphoreType.DMA,
            pltpu.VMEM_SHARED((NS, 16), jnp.float32),
        ],
    )


NS8 = NS // 8
_R8 = NS8 // 2
_LM = NS // 128


def _dinv_body(deg_ref, dinv_ref):
    rid = (lax.broadcasted_iota(jnp.int32, (_LM, 128), 0) * 128
           + lax.broadcasted_iota(jnp.int32, (_LM, 128), 1))
    deg = deg_ref[0] + deg_ref[1] + 1.0
    dinv_ref[...] = jnp.where(rid < N, lax.rsqrt(deg), 0.0)


def _b1_body(t_ref, w_ref, b_ref, rep_ref, out_ref):
    h = jnp.maximum(
        jnp.dot(t_ref[...], w_ref[...], preferred_element_type=jnp.float32)
        + b_ref[...], 0.0)
    out_ref[...] = h * rep_ref[...]


def _b2_body(t_ref, w1_ref, b1_ref, rep_ref, w2_ref, out_ref):
    h2 = jnp.maximum(
        jnp.dot(t_ref[...], w1_ref[...], preferred_element_type=jnp.float32)
        + b1_ref[...], 0.0)
    out_ref[...] = jnp.dot(h2 * rep_ref[...], w2_ref[...],
                           preferred_element_type=jnp.float32)


def _b3_body(a_ref, m_ref, rep_ref, b_ref, out_ref):
    out_ref[...] = (rep_ref[...] * (a_ref[0] + a_ref[1] + m_ref[...])
                    + b_ref[...])


def _blk_spec(d):
    return pl.BlockSpec((_R8, d), lambda i: (i, 0))


def _full_spec(shape):
    return pl.BlockSpec(shape, lambda i: tuple(0 for _ in shape))


def kernel(x, edge_index, W0, b0, W1, b1, W2, b2):
    f32 = jnp.float32
    src = edge_index[0].astype(jnp.int32)
    dst = edge_index[1].astype(jnp.int32)
    pad_vals = N + (jnp.arange(EP - E, dtype=jnp.int32) % 48)
    srcP = jnp.concatenate([src, pad_vals]).reshape(NCH, 128)
    dstP = jnp.concatenate([dst, pad_vals]).reshape(NCH, 128)
    x_pad = jnp.pad(x, ((0, NS - N), (0, 0)))
    zeros16 = jnp.zeros((RPT, 16), f32)
    zeros1 = jnp.zeros((RPT,), f32)
    ones128 = jnp.ones((128,), f32)
    eye8 = jnp.eye(8, dtype=f32)
    W0b = jnp.kron(eye8, W0)
    W1b = jnp.kron(eye8, W1)
    W2b = jnp.kron(eye8, jnp.pad(W2, ((0, 0), (0, 1))))
    b0r = jnp.tile(b0, 8)[None, :]
    b1r = jnp.tile(b1, 8)[None, :]
    b2r = jnp.tile(jnp.pad(b2, (0, 1)), 8)[None, :]

    deg1d = _make_deg()(dstP, ones128, zeros1)
    dinv_lm = pl.pallas_call(
        _dinv_body,
        out_shape=jax.ShapeDtypeStruct((_LM, 128), f32),
    )(deg1d.reshape(2, _LM, 128))
    dinv1 = dinv_lm.reshape(NS)
    d8 = dinv1.reshape(NS8, 8)
    rep512 = jnp.repeat(d8, 64, axis=1)
    rep128 = jnp.repeat(d8, 16, axis=1)

    xsp = x_pad.reshape(NS8, 128) * rep128
    agg0 = _make_agg_edge()(xsp.reshape(NS, 16), srcP, dstP, zeros16)

    a0p = agg0.reshape(2, NS8, 128)
    tin = (a0p[0] + a0p[1] + xsp) * rep128
    hsp = pl.pallas_call(
        _b1_body,
        grid=(2,),
        in_specs=[_blk_spec(128), _full_spec((128, 512)),
                  _full_spec((1, 512)), _blk_spec(512)],
        out_specs=_blk_spec(512),
        out_shape=jax.ShapeDtypeStruct((NS8, 512), f32),
    )(tin, W0b, b0r, rep512)

    agg1 = _make_agg_quad()(hsp.reshape(4 * NS, 16), srcP, dstP, zeros16)

    t1 = (jnp.swapaxes(agg1, 0, 1).reshape(NS8, 512) + hsp) * rep512
    mp = pl.pallas_call(
        _b2_body,
        grid=(2,),
        in_specs=[_blk_spec(512), _full_spec((512, 512)),
                  _full_spec((1, 512)), _blk_spec(512),
                  _full_spec((512, 128))],
        out_specs=_blk_spec(128),
        out_shape=jax.ShapeDtypeStruct((NS8, 128), f32),
    )(t1, W1b, b1r, rep512, W2b)

    agg2 = _make_agg_edge()(mp.reshape(NS, 16), srcP, dstP, zeros16)

    outp = pl.pallas_call(
        _b3_body,
        grid=(2,),
        in_specs=[pl.BlockSpec((2, _R8, 128), lambda i: (0, i, 0)),
                  _blk_spec(128), _blk_spec(128), _full_spec((1, 128))],
        out_specs=_blk_spec(128),
        out_shape=jax.ShapeDtypeStruct((NS8, 128), f32),
    )(agg2.reshape(2, NS8, 128), mp, rep128, b2r)

    return outp.reshape(NS, 16)[:N, :15]
